# Initial kernel scaffold; baseline (speedup 1.0000x reference)
import jax
import jax.numpy as jnp
from jax.experimental import pallas as pl


def kernel(y_pred, length, event, rand_mat):
    raise NotImplementedError("write your pallas kernel here")


# speedup vs baseline: 18.1527x; 18.1527x over previous
import jax
import jax.numpy as jnp
from jax.experimental import pallas as pl
from jax.experimental.pallas import tpu as pltpu

_TOP_N = 2
_REG_WEIGHT = 0.05
_BC = 128


def _cox_kernel(randt_ref, yt_ref, lent_ref, evt_ref, out_ref):
    n = yt_ref.shape[0]
    bc = yt_ref.shape[1]

    y_t = yt_ref[...]
    len_t = lent_ref[...]
    ev_t = evt_ref[...]
    rand3 = randt_ref[...].reshape(n, n, bc)

    maxy = jnp.max(y_t, axis=0, keepdims=True)
    eexp = jnp.exp(y_t - maxy)
    eexp_j = eexp.reshape(1, n, bc)

    len_j = len_t.reshape(1, n, bc)
    len_i = len_t.reshape(n, 1, bc)
    evt_i = ev_t.reshape(n, 1, bc)
    y_i = y_t.reshape(n, 1, bc)

    pair = jnp.where(len_j - len_i > 0.0, evt_i, 0.0)

    if _TOP_N > 0:
        p = pair * (1.0 + rand3)
        work = p
        thr = jnp.zeros((n, 1, bc), jnp.float32)
        for t in range(_TOP_N + 1):
            thr = jnp.max(work, axis=1, keepdims=True)
            if t < _TOP_N:
                work = jnp.where(work == thr, -1.0, work)
        pair = jnp.where(p > thr, pair, 0.0)

    row_sum = jnp.sum(pair, axis=1, keepdims=True)
    valid = row_sum != 0.0

    i_ids = jax.lax.broadcasted_iota(jnp.int32, (n, n, 1), 0)
    j_ids = jax.lax.broadcasted_iota(jnp.int32, (n, n, 1), 1)
    pair = jnp.where((i_ids == j_ids) & valid, 1.0, pair)

    exp_sum = jnp.sum(eexp_j * pair, axis=1, keepdims=True)
    exp_sum = jnp.where(valid, exp_sum, 1.0)
    per_i = jnp.where(valid,
                      maxy.reshape(1, 1, bc) - y_i + jnp.log(exp_sum),
                      0.0)

    loss_main = jnp.sum(per_i, axis=0)
    cols_sum = jnp.sum(pair, axis=0)
    reg = jnp.sum(jnp.abs(cols_sum * y_t), axis=0, keepdims=True)
    out_ref[...] = loss_main + _REG_WEIGHT * reg


@jax.jit
def _cox_batched(y_pred, length, event, rand_mat):
    B, n = rand_mat.shape[0], rand_mat.shape[-1]
    y = y_pred.reshape(B, n).astype(jnp.float32)
    ln = length.reshape(B, n).astype(jnp.float32)
    ev = event.reshape(B, n).astype(jnp.float32)
    rnd = rand_mat.astype(jnp.float32)

    bc = min(_BC, B)
    while B % bc:
        bc -= 1
    c = B // bc

    randt = rnd.reshape(B, n * n).T
    yt = y.T
    lent = ln.T
    evt = ev.T

    out = pl.pallas_call(
        _cox_kernel,
        out_shape=jax.ShapeDtypeStruct((1, B), jnp.float32),
        grid=(c,),
        in_specs=[
            pl.BlockSpec((n * n, bc), lambda i: (0, i)),
            pl.BlockSpec((n, bc), lambda i: (0, i)),
            pl.BlockSpec((n, bc), lambda i: (0, i)),
            pl.BlockSpec((n, bc), lambda i: (0, i)),
        ],
        out_specs=pl.BlockSpec((1, bc), lambda i: (0, i)),
        compiler_params=pltpu.CompilerParams(
            dimension_semantics=("parallel",)),
    )(randt, yt, lent, evt)
    return out.reshape(B)


def kernel(y_pred, length, event, rand_mat):
    return _cox_batched(y_pred, length, event, rand_mat)


# speedup vs baseline: 21.6666x; 1.1936x over previous
import jax
import jax.numpy as jnp
from jax.experimental import pallas as pl
from jax.experimental.pallas import tpu as pltpu

_TOP_N = 2
_REG_WEIGHT = 0.05
_BC = 1024

_TRANS_DIMS = (((0,), (0,)), ((), ()))


_CHUNK = 512


def _cox_kernel(rand_ref, y_ref, len_ref, ev_ref, out_ref):
    bc, nsq = rand_ref.shape
    n = y_ref.shape[1]
    ck = min(_CHUNK, bc)

    eye = (jax.lax.broadcasted_iota(jnp.int32, (ck, ck), 0)
           == jax.lax.broadcasted_iota(jnp.int32, (ck, ck), 1)
           ).astype(jnp.float32)

    def tr(x):
        return jnp.transpose(x)

    sel = (jax.lax.broadcasted_iota(jnp.int32, (n, n * n), 1) // n
           == jax.lax.broadcasted_iota(jnp.int32, (n, n * n), 0)
           ).astype(jnp.float32)

    for s in range(bc // ck):
        lo = s * ck
        y_t = tr(y_ref[pl.ds(lo, ck), :])
        len_t = tr(len_ref[pl.ds(lo, ck), :])
        ev_t = tr(ev_ref[pl.ds(lo, ck), :])
        rand3 = tr(rand_ref[pl.ds(lo, ck), :]).reshape(n, n, ck)

        maxy = jnp.max(y_t, axis=0, keepdims=True)
        eexp = jnp.exp(y_t - maxy)

        len_j = len_t.reshape(1, n, ck)
        len_i = len_t.reshape(n, 1, ck)
        evt_i = ev_t.reshape(n, 1, ck)

        pair = jnp.where(len_j - len_i > 0.0, evt_i, 0.0)

        p = pair * (1.0 + rand3)
        m1 = jnp.max(p, axis=1, keepdims=True)
        keep = p == m1
        for t in range(1, _TOP_N):
            m1 = jnp.max(jnp.where(keep, -1.0, p), axis=1, keepdims=True)
            keep = keep | (p == m1)
        kept = (keep & (p > 0.0)).astype(jnp.float32)

        kflat = kept.reshape(n * n, ck)
        wflat = (kept * eexp.reshape(1, n, ck)).reshape(n * n, ck)
        row_sum = jnp.dot(sel, kflat, preferred_element_type=jnp.float32,
                          precision=jax.lax.Precision.HIGHEST)
        exp_dot = jnp.dot(sel, wflat, preferred_element_type=jnp.float32,
                          precision=jax.lax.Precision.HIGHEST)

        valid = row_sum != 0.0
        valid_f = valid.astype(jnp.float32)
        exp_sum = jnp.where(valid, exp_dot + valid_f * eexp, 1.0)
        per_i = jnp.where(valid, maxy - y_t + jnp.log(exp_sum), 0.0)

        loss_main = jnp.sum(per_i, axis=0, keepdims=True)
        cols_sum = jnp.sum(kept, axis=0) + valid_f
        reg = jnp.sum(jnp.abs(cols_sum * y_t), axis=0, keepdims=True)
        out_ref[0:1, pl.ds(lo, ck)] = loss_main + _REG_WEIGHT * reg


@jax.jit
def _cox_batched(y_pred, length, event, rand_mat):
    B, n = rand_mat.shape[0], rand_mat.shape[-1]
    y = y_pred.reshape(B, n).astype(jnp.float32)
    ln = length.reshape(B, n).astype(jnp.float32)
    ev = event.reshape(B, n).astype(jnp.float32)
    rnd = rand_mat.astype(jnp.float32).reshape(B, n * n)

    bc = min(_BC, B)
    while B % bc:
        bc -= 1
    c = B // bc

    out = pl.pallas_call(
        _cox_kernel,
        out_shape=jax.ShapeDtypeStruct((1, B), jnp.float32),
        grid=(c,),
        in_specs=[
            pl.BlockSpec((bc, n * n), lambda i: (i, 0)),
            pl.BlockSpec((bc, n), lambda i: (i, 0)),
            pl.BlockSpec((bc, n), lambda i: (i, 0)),
            pl.BlockSpec((bc, n), lambda i: (i, 0)),
        ],
        out_specs=pl.BlockSpec((1, bc), lambda i: (0, i)),
        compiler_params=pltpu.CompilerParams(
            dimension_semantics=("parallel",)),
    )(rnd, y, ln, ev)
    return out.reshape(B)


def kernel(y_pred, length, event, rand_mat):
    return _cox_batched(y_pred, length, event, rand_mat)


# speedup vs baseline: 29.9463x; 1.3821x over previous
import jax
import jax.numpy as jnp
from jax.experimental import pallas as pl
from jax.experimental.pallas import tpu as pltpu

_TOP_N = 2
_REG_WEIGHT = 0.05
_BC = 1024

_TRANS_DIMS = (((0,), (0,)), ((), ()))


_CHUNK = 512


def _cox_kernel(rand_ref, y_ref, len_ref, ev_ref, out_ref):
    bc, nsq = rand_ref.shape
    n = y_ref.shape[0]
    ck = min(_CHUNK, bc)

    eye = (jax.lax.broadcasted_iota(jnp.int32, (ck, ck), 0)
           == jax.lax.broadcasted_iota(jnp.int32, (ck, ck), 1)
           ).astype(jnp.float32)

    def tr(x):
        return jnp.transpose(x)

    sel = (jax.lax.broadcasted_iota(jnp.int32, (n, n * n), 1) // n
           == jax.lax.broadcasted_iota(jnp.int32, (n, n * n), 0)
           ).astype(jnp.float32)

    for s in range(bc // ck):
        lo = s * ck
        y_t = y_ref[:, pl.ds(lo, ck)]
        len_t = len_ref[:, pl.ds(lo, ck)]
        ev_t = ev_ref[:, pl.ds(lo, ck)]
        rand3 = tr(rand_ref[pl.ds(lo, ck), :]).reshape(n, n, ck)

        maxy = jnp.max(y_t, axis=0, keepdims=True)
        eexp = jnp.exp(y_t - maxy)

        len_j = len_t.reshape(1, n, ck)
        len_i = len_t.reshape(n, 1, ck)
        evt_i = ev_t.reshape(n, 1, ck)

        pair = jnp.where(len_j - len_i > 0.0, evt_i, 0.0)

        p = pair * (1.0 + rand3)
        m1 = jnp.max(p, axis=1, keepdims=True)
        keep = p == m1
        for t in range(1, _TOP_N):
            m1 = jnp.max(jnp.where(keep, -1.0, p), axis=1, keepdims=True)
            keep = keep | (p == m1)
        kept = (keep & (p > 0.0)).astype(jnp.float32)

        kflat = kept.reshape(n * n, ck)
        wflat = (kept * eexp.reshape(1, n, ck)).reshape(n * n, ck)
        row_sum = jnp.dot(sel, kflat, preferred_element_type=jnp.float32,
                          precision=jax.lax.Precision.HIGHEST)
        exp_dot = jnp.dot(sel, wflat, preferred_element_type=jnp.float32,
                          precision=jax.lax.Precision.HIGHEST)

        valid = row_sum != 0.0
        valid_f = valid.astype(jnp.float32)
        exp_sum = jnp.where(valid, exp_dot + valid_f * eexp, 1.0)
        per_i = jnp.where(valid, maxy - y_t + jnp.log(exp_sum), 0.0)

        loss_main = jnp.sum(per_i, axis=0, keepdims=True)
        cols_sum = jnp.sum(kept, axis=0) + valid_f
        reg = jnp.sum(jnp.abs(cols_sum * y_t), axis=0, keepdims=True)
        out_ref[0:1, pl.ds(lo, ck)] = loss_main + _REG_WEIGHT * reg


@jax.jit
def _cox_batched(y_pred, length, event, rand_mat):
    B, n = rand_mat.shape[0], rand_mat.shape[-1]
    y = y_pred.reshape(B, n).astype(jnp.float32).T
    ln = length.reshape(B, n).astype(jnp.float32).T
    ev = event.reshape(B, n).astype(jnp.float32).T
    rnd = rand_mat.astype(jnp.float32).reshape(B, n * n)

    bc = min(_BC, B)
    while B % bc:
        bc -= 1
    c = B // bc

    out = pl.pallas_call(
        _cox_kernel,
        out_shape=jax.ShapeDtypeStruct((1, B), jnp.float32),
        grid=(c,),
        in_specs=[
            pl.BlockSpec((bc, n * n), lambda i: (i, 0)),
            pl.BlockSpec((n, bc), lambda i: (0, i)),
            pl.BlockSpec((n, bc), lambda i: (0, i)),
            pl.BlockSpec((n, bc), lambda i: (0, i)),
        ],
        out_specs=pl.BlockSpec((1, bc), lambda i: (0, i)),
        compiler_params=pltpu.CompilerParams(
            dimension_semantics=("parallel",)),
    )(rnd, y, ln, ev)
    return out.reshape(B)


def kernel(y_pred, length, event, rand_mat):
    return _cox_batched(y_pred, length, event, rand_mat)
